# Initial kernel scaffold; baseline (speedup 1.0000x reference)
import jax
import jax.numpy as jnp
from jax.experimental import pallas as pl


def kernel(x, embedding, w1, b1, w2, b2, w3, b3):
    raise NotImplementedError("write your pallas kernel here")


# speedup vs baseline: 1.1628x; 1.1628x over previous
import functools

import jax
import jax.numpy as jnp
from jax import lax
from jax.experimental import pallas as pl
from jax.experimental.pallas import tpu as pltpu
from jax.experimental.pallas import tpu_sc as plsc

VOCAB = 100000
EDIM = 300
DPAD = 304
BATCH = 4096
HIST = 50
L = 16
NCHUNK = DPAD // L


def _sc_pool(x_i32, emb_pad):
  mesh = plsc.VectorSubcoreMesh(core_axis_name="c", subcore_axis_name="s",
                                num_cores=2, num_subcores=16)
  nw = mesh.num_cores * mesh.num_subcores
  b_per_w = BATCH // nw

  @functools.partial(
      pl.kernel,
      out_type=jax.ShapeDtypeStruct((BATCH, DPAD), jnp.float32),
      mesh=mesh,
      scratch_types=[
          pltpu.VMEM((b_per_w, HIST), jnp.int32),
          pltpu.VMEM((HIST, DPAD), jnp.float32),
          pltpu.VMEM((b_per_w, DPAD), jnp.float32),
          pltpu.SemaphoreType.DMA,
      ],
      compiler_params=pltpu.CompilerParams(use_tc_tiling_on_sc=False),
  )
  def pool(emb_hbm, x_hbm, out_hbm, idx_v, rows_v, stage_v, sem):
    wid = lax.axis_index("s") * mesh.num_cores + lax.axis_index("c")
    base = wid * b_per_w
    pltpu.sync_copy(x_hbm.at[pl.ds(base, b_per_w), :], idx_v)

    inv = jnp.full((L,), 1.0 / HIST, dtype=jnp.float32)
    zeros = jnp.zeros((L,), dtype=jnp.float32)

    def per_sample(s, carry):
      pltpu.async_copy(emb_hbm.at[idx_v.at[s]], rows_v, sem).wait()

      def accum(r, accs):
        return tuple(accs[c] + rows_v[r, pl.ds(c * L, L)]
                     for c in range(NCHUNK))

      accs = lax.fori_loop(0, HIST, accum, (zeros,) * NCHUNK)
      for c in range(NCHUNK):
        stage_v[s, pl.ds(c * L, L)] = accs[c] * inv
      return carry

    lax.fori_loop(0, b_per_w, per_sample, 0)
    pltpu.sync_copy(stage_v, out_hbm.at[pl.ds(base, b_per_w), :])

  return pool(emb_pad, x_i32)


def _mlp_block(p_ref, w1_ref, b1_ref, w2_ref, b2_ref, w3_ref, b3_ref, o_ref):
  h = jnp.dot(p_ref[...], w1_ref[...], preferred_element_type=jnp.float32)
  h = jnp.maximum(h + b1_ref[...], 0.0)
  h = jnp.dot(h, w2_ref[...], preferred_element_type=jnp.float32)
  h = jnp.maximum(h + b2_ref[...], 0.0)
  o_ref[...] = (
      jnp.dot(h, w3_ref[...], preferred_element_type=jnp.float32)
      + b3_ref[...])


def _tc_mlp(pooled, w1p, b1, w2, b2, w3, b3):
  bb = 1024
  grid = (BATCH // bb,)
  return pl.pallas_call(
      _mlp_block,
      grid=grid,
      in_specs=[
          pl.BlockSpec((bb, DPAD), lambda i: (i, 0)),
          pl.BlockSpec((DPAD, 150), lambda i: (0, 0)),
          pl.BlockSpec((1, 150), lambda i: (0, 0)),
          pl.BlockSpec((150, 150), lambda i: (0, 0)),
          pl.BlockSpec((1, 150), lambda i: (0, 0)),
          pl.BlockSpec((150, 1), lambda i: (0, 0)),
          pl.BlockSpec((1, 1), lambda i: (0, 0)),
      ],
      out_specs=pl.BlockSpec((bb, 1), lambda i: (i, 0)),
      out_shape=jax.ShapeDtypeStruct((BATCH, 1), jnp.float32),
  )(pooled, w1p, b1, w2, b2, w3, b3)


def kernel(x, embedding, w1, b1, w2, b2, w3, b3):
  x_i32 = x.astype(jnp.int32)
  emb_pad = jnp.pad(embedding, ((0, 0), (0, DPAD - EDIM)))
  pooled = _sc_pool(x_i32, emb_pad)
  w1p = jnp.pad(w1, ((0, DPAD - EDIM), (0, 0)))
  return _tc_mlp(pooled, w1p, b1.reshape(1, 150), w2, b2.reshape(1, 150),
                 w3, b3.reshape(1, 1))


# speedup vs baseline: 2.6861x; 2.3101x over previous
import functools

import jax
import jax.numpy as jnp
from jax import lax
from jax.experimental import pallas as pl
from jax.experimental.pallas import tpu as pltpu
from jax.experimental.pallas import tpu_sc as plsc

VOCAB = 100000
EDIM = 300
HID = 150
HPAD = 160
BATCH = 4096
HIST = 50
L = 16
NCHUNK = HPAD // L


def _proj_block(emb_ref, w1_ref, o_ref):
  o_ref[...] = jnp.dot(emb_ref[...], w1_ref[...],
                       preferred_element_type=jnp.float32)


def _tc_project(emb, w1p):
  bp = 2000
  return pl.pallas_call(
      _proj_block,
      grid=(VOCAB // bp,),
      in_specs=[
          pl.BlockSpec((bp, EDIM), lambda i: (i, 0)),
          pl.BlockSpec((EDIM, HPAD), lambda i: (0, 0)),
      ],
      out_specs=pl.BlockSpec((bp, HPAD), lambda i: (i, 0)),
      out_shape=jax.ShapeDtypeStruct((VOCAB, HPAD), jnp.float32),
  )(emb, w1p)


def _sc_pool(x_i32, proj):
  mesh = plsc.VectorSubcoreMesh(core_axis_name="c", subcore_axis_name="s",
                                num_cores=2, num_subcores=16)
  nw = mesh.num_cores * mesh.num_subcores
  b_per_w = BATCH // nw

  @functools.partial(
      pl.kernel,
      out_type=jax.ShapeDtypeStruct((BATCH, HPAD), jnp.float32),
      mesh=mesh,
      scratch_types=[
          pltpu.VMEM((b_per_w, HIST), jnp.int32),
          pltpu.VMEM((HIST, HPAD), jnp.float32),
          pltpu.VMEM((HIST, HPAD), jnp.float32),
          pltpu.VMEM((b_per_w, HPAD), jnp.float32),
          pltpu.SemaphoreType.DMA,
          pltpu.SemaphoreType.DMA,
      ],
      compiler_params=pltpu.CompilerParams(use_tc_tiling_on_sc=False),
  )
  def pool(proj_hbm, x_hbm, out_hbm, idx_v, rows0_v, rows1_v, stage_v,
           sem0, sem1):
    wid = lax.axis_index("s") * mesh.num_cores + lax.axis_index("c")
    base = wid * b_per_w
    pltpu.sync_copy(x_hbm.at[pl.ds(base, b_per_w), :], idx_v)

    inv = jnp.full((L,), 1.0 / HIST, dtype=jnp.float32)
    rows = (rows0_v, rows1_v)
    sems = (sem0, sem1)

    pltpu.async_copy(proj_hbm.at[idx_v.at[0]], rows0_v, sem0)

    def per_sample(s, carry):
      parity = lax.rem(s, 2)

      @pl.when(s + 1 < b_per_w)
      def _():
        def start(p):
          pltpu.async_copy(proj_hbm.at[idx_v.at[s + 1]], rows[p], sems[p])
        lax.cond(parity == 0, lambda: start(1), lambda: start(0))

      def finish(p):
        pltpu.make_async_copy(proj_hbm.at[idx_v.at[s]], rows[p],
                              sems[p]).wait()

        def accum(r, accs):
          return tuple(accs[c] + rows[p][r, pl.ds(c * L, L)]
                       for c in range(NCHUNK))

        accs = lax.fori_loop(0, HIST, accum,
                             (jnp.zeros((L,), jnp.float32),) * NCHUNK)
        for c in range(NCHUNK):
          stage_v[s, pl.ds(c * L, L)] = accs[c] * inv

      lax.cond(parity == 0, lambda: finish(0), lambda: finish(1))
      return carry

    lax.fori_loop(0, b_per_w, per_sample, 0)
    pltpu.sync_copy(stage_v, out_hbm.at[pl.ds(base, b_per_w), :])

  return pool(proj, x_i32)


def _mlp_block(p_ref, b1_ref, w2_ref, b2_ref, w3_ref, b3_ref, o_ref):
  h = jnp.maximum(p_ref[...] + b1_ref[...], 0.0)
  h = jnp.dot(h, w2_ref[...], preferred_element_type=jnp.float32)
  h = jnp.maximum(h + b2_ref[...], 0.0)
  o_ref[...] = (
      jnp.dot(h, w3_ref[...], preferred_element_type=jnp.float32)
      + b3_ref[...])


def _tc_mlp(pooled, b1p, w2p, b2, w3, b3):
  bb = 1024
  return pl.pallas_call(
      _mlp_block,
      grid=(BATCH // bb,),
      in_specs=[
          pl.BlockSpec((bb, HPAD), lambda i: (i, 0)),
          pl.BlockSpec((1, HPAD), lambda i: (0, 0)),
          pl.BlockSpec((HPAD, HID), lambda i: (0, 0)),
          pl.BlockSpec((1, HID), lambda i: (0, 0)),
          pl.BlockSpec((HID, 1), lambda i: (0, 0)),
          pl.BlockSpec((1, 1), lambda i: (0, 0)),
      ],
      out_specs=pl.BlockSpec((bb, 1), lambda i: (i, 0)),
      out_shape=jax.ShapeDtypeStruct((BATCH, 1), jnp.float32),
  )(pooled, b1p, w2p, b2, w3, b3)


def kernel(x, embedding, w1, b1, w2, b2, w3, b3):
  x_i32 = x.astype(jnp.int32)
  w1p = jnp.pad(w1, ((0, 0), (0, HPAD - HID)))
  proj = _tc_project(embedding, w1p)
  pooled = _sc_pool(x_i32, proj)
  b1p = jnp.pad(b1, (0, HPAD - HID)).reshape(1, HPAD)
  w2p = jnp.pad(w2, ((0, HPAD - HID), (0, 0)))
  return _tc_mlp(pooled, b1p, w2p, b2.reshape(1, HID), w3,
                 b3.reshape(1, 1))


# speedup vs baseline: 2.6988x; 1.0047x over previous
import functools

import jax
import jax.numpy as jnp
from jax import lax
from jax.experimental import pallas as pl
from jax.experimental.pallas import tpu as pltpu
from jax.experimental.pallas import tpu_sc as plsc

VOCAB = 100000
EDIM = 300
HID = 150
HPAD = 160
BATCH = 4096
HIST = 50
L = 16
NCHUNK = HPAD // L


def _proj_block(emb_ref, w1_ref, o_ref):
  o_ref[...] = jnp.dot(emb_ref[...].astype(jnp.bfloat16),
                       w1_ref[...].astype(jnp.bfloat16),
                       preferred_element_type=jnp.float32)


def _tc_project(emb, w1p):
  bp = 2000
  return pl.pallas_call(
      _proj_block,
      grid=(VOCAB // bp,),
      in_specs=[
          pl.BlockSpec((bp, EDIM), lambda i: (i, 0)),
          pl.BlockSpec((EDIM, HPAD), lambda i: (0, 0)),
      ],
      out_specs=pl.BlockSpec((bp, HPAD), lambda i: (i, 0)),
      out_shape=jax.ShapeDtypeStruct((VOCAB, HPAD), jnp.float32),
  )(emb, w1p)


def _sc_pool(x_i32, proj):
  mesh = plsc.VectorSubcoreMesh(core_axis_name="c", subcore_axis_name="s",
                                num_cores=2, num_subcores=16)
  nw = mesh.num_cores * mesh.num_subcores
  b_per_w = BATCH // nw

  @functools.partial(
      pl.kernel,
      out_type=jax.ShapeDtypeStruct((BATCH, HPAD), jnp.float32),
      mesh=mesh,
      scratch_types=[
          pltpu.VMEM((b_per_w, HIST), jnp.int32),
          pltpu.VMEM((HIST, HPAD), jnp.float32),
          pltpu.VMEM((HIST, HPAD), jnp.float32),
          pltpu.VMEM((b_per_w, HPAD), jnp.float32),
          pltpu.SemaphoreType.DMA,
          pltpu.SemaphoreType.DMA,
      ],
      compiler_params=pltpu.CompilerParams(use_tc_tiling_on_sc=False),
  )
  def pool(proj_hbm, x_hbm, out_hbm, idx_v, rows0_v, rows1_v, stage_v,
           sem0, sem1):
    wid = lax.axis_index("s") * mesh.num_cores + lax.axis_index("c")
    base = wid * b_per_w
    pltpu.sync_copy(x_hbm.at[pl.ds(base, b_per_w), :], idx_v)

    inv = jnp.full((L,), 1.0 / HIST, dtype=jnp.float32)
    rows = (rows0_v, rows1_v)
    sems = (sem0, sem1)

    pltpu.async_copy(proj_hbm.at[idx_v.at[0]], rows0_v, sem0)

    def per_sample(s, carry):
      parity = lax.rem(s, 2)

      @pl.when(s + 1 < b_per_w)
      def _():
        def start(p):
          pltpu.async_copy(proj_hbm.at[idx_v.at[s + 1]], rows[p], sems[p])
        lax.cond(parity == 0, lambda: start(1), lambda: start(0))

      def finish(p):
        pltpu.make_async_copy(proj_hbm.at[idx_v.at[s]], rows[p],
                              sems[p]).wait()

        def accum(r, accs):
          return tuple(accs[c] + rows[p][r, pl.ds(c * L, L)]
                       for c in range(NCHUNK))

        accs = lax.fori_loop(0, HIST, accum,
                             (jnp.zeros((L,), jnp.float32),) * NCHUNK)
        for c in range(NCHUNK):
          stage_v[s, pl.ds(c * L, L)] = accs[c] * inv

      lax.cond(parity == 0, lambda: finish(0), lambda: finish(1))
      return carry

    lax.fori_loop(0, b_per_w, per_sample, 0)
    pltpu.sync_copy(stage_v, out_hbm.at[pl.ds(base, b_per_w), :])

  return pool(proj, x_i32)


def _mlp_block(p_ref, b1_ref, w2_ref, b2_ref, w3_ref, b3_ref, o_ref):
  h = jnp.maximum(p_ref[...] + b1_ref[...], 0.0)
  h = jnp.dot(h, w2_ref[...], preferred_element_type=jnp.float32)
  h = jnp.maximum(h + b2_ref[...], 0.0)
  o_ref[...] = (
      jnp.dot(h, w3_ref[...], preferred_element_type=jnp.float32)
      + b3_ref[...])


def _tc_mlp(pooled, b1p, w2p, b2, w3, b3):
  bb = 1024
  return pl.pallas_call(
      _mlp_block,
      grid=(BATCH // bb,),
      in_specs=[
          pl.BlockSpec((bb, HPAD), lambda i: (i, 0)),
          pl.BlockSpec((1, HPAD), lambda i: (0, 0)),
          pl.BlockSpec((HPAD, HID), lambda i: (0, 0)),
          pl.BlockSpec((1, HID), lambda i: (0, 0)),
          pl.BlockSpec((HID, 1), lambda i: (0, 0)),
          pl.BlockSpec((1, 1), lambda i: (0, 0)),
      ],
      out_specs=pl.BlockSpec((bb, 1), lambda i: (i, 0)),
      out_shape=jax.ShapeDtypeStruct((BATCH, 1), jnp.float32),
  )(pooled, b1p, w2p, b2, w3, b3)


def kernel(x, embedding, w1, b1, w2, b2, w3, b3):
  x_i32 = x.astype(jnp.int32)
  w1p = jnp.pad(w1, ((0, 0), (0, HPAD - HID)))
  proj = _tc_project(embedding, w1p)
  pooled = _sc_pool(x_i32, proj)
  b1p = jnp.pad(b1, (0, HPAD - HID)).reshape(1, HPAD)
  w2p = jnp.pad(w2, ((0, HPAD - HID), (0, 0)))
  return _tc_mlp(pooled, b1p, w2p, b2.reshape(1, HID), w3,
                 b3.reshape(1, 1))


# speedup vs baseline: 3.7933x; 1.4055x over previous
import functools

import jax
import jax.numpy as jnp
from jax import lax
from jax.experimental import pallas as pl
from jax.experimental.pallas import tpu as pltpu
from jax.experimental.pallas import tpu_sc as plsc

VOCAB = 100000
EDIM = 300
HID = 150
HPAD = 160
BATCH = 4096
HIST = 50
L = 16
NCHUNK = HPAD // L


def _proj_block(embT_ref, w1_ref, o_ref):
  o_ref[...] = lax.dot_general(
      embT_ref[...].astype(jnp.bfloat16),
      w1_ref[...].astype(jnp.bfloat16),
      dimension_numbers=(((0,), (0,)), ((), ())),
      preferred_element_type=jnp.float32)


def _tc_project(embT, w1p):
  bp = 2048
  nblk = -(-VOCAB // bp)
  return pl.pallas_call(
      _proj_block,
      grid=(nblk,),
      in_specs=[
          pl.BlockSpec((EDIM, bp), lambda i: (0, i)),
          pl.BlockSpec((EDIM, HPAD), lambda i: (0, 0)),
      ],
      out_specs=pl.BlockSpec((bp, HPAD), lambda i: (i, 0)),
      out_shape=jax.ShapeDtypeStruct((nblk * bp, HPAD), jnp.float32),
  )(embT, w1p)


def _sc_pool(x_i32, proj):
  mesh = plsc.VectorSubcoreMesh(core_axis_name="c", subcore_axis_name="s",
                                num_cores=2, num_subcores=16)
  nw = mesh.num_cores * mesh.num_subcores
  b_per_w = BATCH // nw

  @functools.partial(
      pl.kernel,
      out_type=jax.ShapeDtypeStruct((BATCH, HPAD), jnp.float32),
      mesh=mesh,
      scratch_types=[
          pltpu.VMEM((b_per_w, HIST), jnp.int32),
          pltpu.VMEM((HIST, HPAD), jnp.float32),
          pltpu.VMEM((HIST, HPAD), jnp.float32),
          pltpu.VMEM((b_per_w, HPAD), jnp.float32),
          pltpu.SemaphoreType.DMA,
          pltpu.SemaphoreType.DMA,
      ],
      compiler_params=pltpu.CompilerParams(use_tc_tiling_on_sc=False),
  )
  def pool(proj_hbm, x_hbm, out_hbm, idx_v, rows0_v, rows1_v, stage_v,
           sem0, sem1):
    wid = lax.axis_index("s") * mesh.num_cores + lax.axis_index("c")
    base = wid * b_per_w
    pltpu.sync_copy(x_hbm.at[pl.ds(base, b_per_w), :], idx_v)

    inv = jnp.full((L,), 1.0 / HIST, dtype=jnp.float32)
    rows = (rows0_v, rows1_v)
    sems = (sem0, sem1)

    pltpu.async_copy(proj_hbm.at[idx_v.at[0]], rows0_v, sem0)

    def per_sample(s, carry):
      parity = lax.rem(s, 2)

      @pl.when(s + 1 < b_per_w)
      def _():
        def start(p):
          pltpu.async_copy(proj_hbm.at[idx_v.at[s + 1]], rows[p], sems[p])
        lax.cond(parity == 0, lambda: start(1), lambda: start(0))

      def finish(p):
        pltpu.make_async_copy(proj_hbm.at[idx_v.at[s]], rows[p],
                              sems[p]).wait()

        def accum(r, accs):
          return tuple(accs[c] + rows[p][r, pl.ds(c * L, L)]
                       for c in range(NCHUNK))

        accs = lax.fori_loop(0, HIST, accum,
                             (jnp.zeros((L,), jnp.float32),) * NCHUNK)
        for c in range(NCHUNK):
          stage_v[s, pl.ds(c * L, L)] = accs[c] * inv

      lax.cond(parity == 0, lambda: finish(0), lambda: finish(1))
      return carry

    lax.fori_loop(0, b_per_w, per_sample, 0)
    pltpu.sync_copy(stage_v, out_hbm.at[pl.ds(base, b_per_w), :])

  return pool(proj, x_i32)


def _mlp_block(p_ref, b1_ref, w2_ref, b2_ref, w3_ref, b3_ref, o_ref):
  h = jnp.maximum(p_ref[...] + b1_ref[...], 0.0)
  h = jnp.dot(h, w2_ref[...], preferred_element_type=jnp.float32)
  h = jnp.maximum(h + b2_ref[...], 0.0)
  o_ref[...] = (
      jnp.dot(h, w3_ref[...], preferred_element_type=jnp.float32)
      + b3_ref[...])


def _tc_mlp(pooled, b1p, w2p, b2, w3, b3):
  bb = 1024
  return pl.pallas_call(
      _mlp_block,
      grid=(BATCH // bb,),
      in_specs=[
          pl.BlockSpec((bb, HPAD), lambda i: (i, 0)),
          pl.BlockSpec((1, HPAD), lambda i: (0, 0)),
          pl.BlockSpec((HPAD, HID), lambda i: (0, 0)),
          pl.BlockSpec((1, HID), lambda i: (0, 0)),
          pl.BlockSpec((HID, 1), lambda i: (0, 0)),
          pl.BlockSpec((1, 1), lambda i: (0, 0)),
      ],
      out_specs=pl.BlockSpec((bb, 1), lambda i: (i, 0)),
      out_shape=jax.ShapeDtypeStruct((BATCH, 1), jnp.float32),
  )(pooled, b1p, w2p, b2, w3, b3)


def kernel(x, embedding, w1, b1, w2, b2, w3, b3):
  x_i32 = x.astype(jnp.int32)
  w1p = jnp.pad(w1, ((0, 0), (0, HPAD - HID)))
  proj = _tc_project(embedding.T, w1p)
  pooled = _sc_pool(x_i32, proj)
  b1p = jnp.pad(b1, (0, HPAD - HID)).reshape(1, HPAD)
  w2p = jnp.pad(w2, ((0, HPAD - HID), (0, 0)))
  return _tc_mlp(pooled, b1p, w2p, b2.reshape(1, HID), w3,
                 b3.reshape(1, 1))
